# baseline (speedup 1.0000x reference)
import functools

import jax
import jax.numpy as jnp
from jax import lax
from jax.experimental import pallas as pl
from jax.experimental.pallas import tpu as pltpu
from jax.experimental.pallas import tpu_sc as plsc

_ROWS, _COLS = 128, 32768
_BLOCK_ROWS = 8
_NW = 32
_GROUPS = _ROWS // 8
_HALF = _COLS // 2
_CHUNK = 2048

_mesh = plsc.VectorSubcoreMesh(
    core_axis_name="c", subcore_axis_name="s", num_cores=2, num_subcores=16
)


def _argmax_body(x_ref, idx_ref):
    xb = x_ref[...]
    m = jnp.max(xb, axis=-1, keepdims=True)
    iota = lax.broadcasted_iota(jnp.int32, xb.shape, 1)
    big = jnp.int32(2**30)
    idx_ref[...] = jnp.min(jnp.where(xb == m, iota, big), axis=-1, keepdims=True)


def _argmax_call(x):
    return pl.pallas_call(
        _argmax_body,
        out_shape=jax.ShapeDtypeStruct((_ROWS, 1), jnp.int32),
        grid=(_ROWS // _BLOCK_ROWS,),
        in_specs=[pl.BlockSpec((_BLOCK_ROWS, _COLS), lambda i: (i, 0))],
        out_specs=pl.BlockSpec((_BLOCK_ROWS, 1), lambda i: (i, 0)),
    )(x)


@functools.partial(
    pl.kernel,
    out_type=jax.ShapeDtypeStruct((_ROWS, _COLS), jnp.float32),
    mesh=_mesh,
    scratch_types=[pltpu.VMEM((8, _CHUNK), jnp.float32)],
)
def _fill(out_hbm, zbuf):
    zero16 = jnp.zeros((16,), jnp.float32)
    for k in range(8):
        @pl.loop(0, _CHUNK // 16, unroll=8)
        def _zero(i, k=k):
            zbuf[k, pl.ds(i * 16, 16)] = zero16

    wid = lax.axis_index("s") * 2 + lax.axis_index("c")
    rowbase = pl.multiple_of((wid // 2) * 8, 8)
    colbase = pl.multiple_of((wid % 2) * _HALF, 128)

    @pl.loop(0, _HALF // _CHUNK)
    def _copy(j):
        col = pl.multiple_of(colbase + j * _CHUNK, 128)
        pltpu.sync_copy(zbuf, out_hbm.at[pl.ds(rowbase, 8), pl.ds(col, _CHUNK)])


@functools.partial(
    pl.kernel,
    out_type=(),
    mesh=_mesh,
    scratch_types=[
        pltpu.VMEM((_ROWS + 16,), jnp.int32),
        pltpu.VMEM((8, 128), jnp.float32),
    ],
)
def _scatter(idx_hbm, buf_ref, idx_v, tile_v):
    pltpu.sync_copy(idx_hbm, idx_v.at[pl.ds(0, _ROWS)])
    wid = lax.axis_index("s") * 2 + lax.axis_index("c")
    zero16 = jnp.zeros((16,), jnp.float32)

    for k in range(8):
        for m in range(8):
            tile_v[k, pl.ds(m * 16, 16)] = zero16

    @pl.when(wid < _GROUPS)
    def _():
        rowbase = pl.multiple_of(wid * 8, 8)
        lane = lax.iota(jnp.int32, 16)
        one_i = jnp.full((16,), 1, jnp.int32)
        vec = idx_v[pl.ds(rowbase, 16)]
        ct = [vec[k] >> 7 for k in range(8)]
        off = [vec[k] & jnp.int32(127) for k in range(8)]

        def onehot16(o):
            d = lane - jnp.full((16,), o, jnp.int32)
            return jnp.maximum(one_i - jnp.abs(d), 0).astype(jnp.float32)

        for j in range(8):
            fresh = jnp.bool_(True)
            for i in range(j):
                fresh = jnp.logical_and(fresh, ct[i] != ct[j])

            @pl.when(fresh)
            def _(j=j):
                for k in range(8):
                    @pl.when(ct[k] == ct[j])
                    def _(k=k):
                        w16 = off[k] & jnp.int32(-16)
                        tile_v[k, pl.ds(w16, 16)] = onehot16(off[k] & jnp.int32(15))

                colbase = pl.multiple_of(ct[j] * 128, 128)
                pltpu.sync_copy(
                    tile_v, buf_ref.at[pl.ds(rowbase, 8), pl.ds(colbase, 128)]
                )
                for k in range(8):
                    @pl.when(ct[k] == ct[j])
                    def _(k=k):
                        w16 = off[k] & jnp.int32(-16)
                        tile_v[k, pl.ds(w16, 16)] = zero16


def kernel(x):
    idx = _argmax_call(x).reshape(_ROWS)
    buf = jax.new_ref(_fill())
    _scatter(idx, buf)
    return buf[...]


# speedup vs baseline: 2.0534x; 2.0534x over previous
import jax
import jax.numpy as jnp
from jax import lax
from jax.experimental import pallas as pl


_ROWS, _COLS = 128, 32768
_BLOCK_ROWS = 8


def _onehot_argmax_kernel(x_ref, o_ref):
    xb = x_ref[...]
    first = jnp.argmax(xb, axis=-1, keepdims=True).astype(jnp.int32)
    iota = lax.broadcasted_iota(jnp.int32, xb.shape, 1)
    o_ref[...] = (iota == first).astype(jnp.float32)


def kernel(x):
    grid = (_ROWS // _BLOCK_ROWS,)
    return pl.pallas_call(
        _onehot_argmax_kernel,
        out_shape=jax.ShapeDtypeStruct((_ROWS, _COLS), jnp.float32),
        grid=grid,
        in_specs=[pl.BlockSpec((_BLOCK_ROWS, _COLS), lambda i: (i, 0))],
        out_specs=pl.BlockSpec((_BLOCK_ROWS, _COLS), lambda i: (i, 0)),
    )(x)


# speedup vs baseline: 2.8238x; 1.3752x over previous
import jax
import jax.numpy as jnp
from jax import lax
from jax.experimental import pallas as pl


_ROWS, _COLS = 128, 32768
_BLOCK_ROWS = 16


def _onehot_argmax_kernel(x_ref, o_ref):
    xb = x_ref[...]
    first = jnp.argmax(xb, axis=-1, keepdims=True).astype(jnp.int32)
    iota = lax.broadcasted_iota(jnp.int32, xb.shape, 1)
    o_ref[...] = (iota == first).astype(jnp.float32)


def kernel(x):
    grid = (_ROWS // _BLOCK_ROWS,)
    return pl.pallas_call(
        _onehot_argmax_kernel,
        out_shape=jax.ShapeDtypeStruct((_ROWS, _COLS), jnp.float32),
        grid=grid,
        in_specs=[pl.BlockSpec((_BLOCK_ROWS, _COLS), lambda i: (i, 0))],
        out_specs=pl.BlockSpec((_BLOCK_ROWS, _COLS), lambda i: (i, 0)),
    )(x)


# speedup vs baseline: 3.0766x; 1.0895x over previous
import jax
import jax.numpy as jnp
from jax import lax
from jax.experimental import pallas as pl


_ROWS, _COLS = 128, 32768
_BLOCK_ROWS = 32


def _onehot_argmax_kernel(x_ref, o_ref):
    xb = x_ref[...]
    first = jnp.argmax(xb, axis=-1, keepdims=True).astype(jnp.int32)
    iota = lax.broadcasted_iota(jnp.int32, xb.shape, 1)
    o_ref[...] = (iota == first).astype(jnp.float32)


def kernel(x):
    grid = (_ROWS // _BLOCK_ROWS,)
    return pl.pallas_call(
        _onehot_argmax_kernel,
        out_shape=jax.ShapeDtypeStruct((_ROWS, _COLS), jnp.float32),
        grid=grid,
        in_specs=[pl.BlockSpec((_BLOCK_ROWS, _COLS), lambda i: (i, 0))],
        out_specs=pl.BlockSpec((_BLOCK_ROWS, _COLS), lambda i: (i, 0)),
    )(x)


# speedup vs baseline: 3.4266x; 1.1138x over previous
import jax
import jax.numpy as jnp
from jax import lax
from jax.experimental import pallas as pl


_ROWS, _COLS = 128, 32768
_BLOCK_ROWS = 64


def _onehot_argmax_kernel(x_ref, o_ref):
    xb = x_ref[...]
    first = jnp.argmax(xb, axis=-1, keepdims=True).astype(jnp.int32)
    iota = lax.broadcasted_iota(jnp.int32, xb.shape, 1)
    o_ref[...] = (iota == first).astype(jnp.float32)


def kernel(x):
    grid = (_ROWS // _BLOCK_ROWS,)
    return pl.pallas_call(
        _onehot_argmax_kernel,
        out_shape=jax.ShapeDtypeStruct((_ROWS, _COLS), jnp.float32),
        grid=grid,
        in_specs=[pl.BlockSpec((_BLOCK_ROWS, _COLS), lambda i: (i, 0))],
        out_specs=pl.BlockSpec((_BLOCK_ROWS, _COLS), lambda i: (i, 0)),
    )(x)
